# Initial kernel scaffold; baseline (speedup 1.0000x reference)
import jax
import jax.numpy as jnp
from jax.experimental import pallas as pl


def kernel(diseases_embed, pros_embed, meds_embed, Wh, bh, Wl, bl, hg_W, hg_b, g_W, g_b, ddi_W, ddi_b, A_val, H_idx, A_idx, ddi_idx):
    raise NotImplementedError("write your pallas kernel here")


# speedup vs baseline: 6.7364x; 6.7364x over previous
import functools

import jax
import jax.numpy as jnp
from jax import lax
from jax.experimental import pallas as pl
from jax.experimental.pallas import tpu as pltpu
from jax.experimental.pallas import tpu_sc as plsc

N_DIS, N_PRO, N_MED = 4000, 4000, 2000
N = N_DIS + N_PRO + N_MED
D = 128
E = 2000
L = 2

NC, NS, LANES = 2, 16, 16
NW = NC * NS
CH = 80


def _mesh():
    return plsc.VectorSubcoreMesh(
        core_axis_name="c", subcore_axis_name="s",
        num_cores=NC, num_subcores=NS)


@functools.lru_cache(maxsize=None)
def _sc_pass(K, mode, R1, R2=0):
    per_w = K // NW
    assert per_w * NW == K and per_w % CH == 0
    nch = per_w // CH

    two_acc = mode in ("seg2", "cnt2")
    gather = mode in ("seg", "segw", "seg2")
    weighted = mode in ("segw", "cntw")
    ones = mode in ("cnt", "cntw", "cnt2")
    Rs = [R1, R2] if mode == "cnt2" else ([R1, R1] if two_acc else [R1])
    nbuf = 2 if mode == "seg2" else 1

    scratch = (
        [pltpu.VMEM((CH,), jnp.int32)] * (2 if mode == "cnt2" else 1)
        + ([pltpu.VMEM((CH,), jnp.int32)] if gather else [])
        + [pltpu.VMEM((CH, D), jnp.float32) for _ in range(nbuf)]
        + [pltpu.VMEM_SHARED((r, D), jnp.float32) for r in Rs]
        + [pltpu.SemaphoreType.DMA for _ in range(nbuf)]
        + ([pltpu.VMEM((CH,), jnp.float32)] if weighted else [])
    )

    def body(*refs):
        refs = list(refs)
        tabs = [refs.pop(0) for _ in range(nbuf)] if gather else []
        dsts = [refs.pop(0) for _ in range(2 if mode == "cnt2" else 1)]
        src_hbm = refs.pop(0) if gather else None
        ew_hbm = refs.pop(0) if weighted else None
        outs = [refs.pop(0) for _ in range(len(Rs))]
        dst_vs = [refs.pop(0) for _ in range(2 if mode == "cnt2" else 1)]
        src_v = refs.pop(0) if gather else None
        rows = [refs.pop(0) for _ in range(nbuf)]
        shareds = [refs.pop(0) for _ in range(len(Rs))]
        sems = [refs.pop(0) for _ in range(nbuf)]
        ew_v = refs.pop(0) if weighted else None

        cid = lax.axis_index("c")
        sid = lax.axis_index("s")
        wid = sid * NC + cid

        zv = jnp.zeros((LANES,), jnp.float32)

        def zfill(j, carry):
            for p in range(D // LANES):
                rows[0][j, pl.ds(p * LANES, LANES)] = zv
            return carry

        lax.fori_loop(0, CH, zfill, 0)

        for shared, r in zip(shareds, Rs):
            nzblk = r // CH

            def zcopy(t, carry, shared=shared, nzblk=nzblk):
                b = t * NS + sid

                @pl.when(b < nzblk)
                def _():
                    off = pl.multiple_of(b * CH, 8)
                    pltpu.sync_copy(rows[0], shared.at[pl.ds(off, CH)])
                return carry

            lax.fori_loop(0, (nzblk + NS - 1) // NS, zcopy, 0)

        if ones and mode != "cntw":
            ov = jnp.ones((LANES,), jnp.float32)

            def ofill(j, carry):
                for p in range(D // LANES):
                    rows[0][j, pl.ds(p * LANES, LANES)] = ov
                return carry

            lax.fori_loop(0, CH, ofill, 0)

        plsc.subcore_barrier()

        base = wid * per_w

        def step(t, carry):
            off = base + t * CH
            for dh, dv in zip(dsts, dst_vs):
                pltpu.sync_copy(dh.at[pl.ds(off, CH)], dv)
            if gather:
                pltpu.sync_copy(src_hbm.at[pl.ds(off, CH)], src_v)
                cps = [pltpu.async_copy(tab.at[src_v], rb, sem)
                       for tab, rb, sem in zip(tabs, rows, sems)]
                for cp in cps:
                    cp.wait()
            if weighted:
                pltpu.sync_copy(ew_hbm.at[pl.ds(off, CH)], ew_v)
                if not gather:
                    ov = jnp.ones((LANES,), jnp.float32)

                    def refill(j, c2):
                        for p in range(D // LANES):
                            rows[0][j, pl.ds(p * LANES, LANES)] = ov
                        return c2

                    lax.fori_loop(0, CH, refill, 0)

                def scale(q, c2):
                    wchunk = ew_v[pl.ds(q * LANES, LANES)]
                    for jj in range(LANES):
                        w = wchunk[jnp.full((LANES,), jj, jnp.int32)]
                        j = q * LANES + jj
                        for p in range(D // LANES):
                            sl = pl.ds(p * LANES, LANES)
                            rows[0][j, sl] = rows[0][j, sl] * w
                    return c2

                lax.fori_loop(0, CH // LANES, scale, 0)
            if mode == "cnt2":
                pltpu.sync_copy(rows[0], shareds[0].at[dst_vs[0]], add=True)
                pltpu.sync_copy(rows[0], shareds[1].at[dst_vs[1]], add=True)
            else:
                for rb, shared in zip(rows, shareds):
                    pltpu.sync_copy(rb, shared.at[dst_vs[0]], add=True)
            return carry

        lax.fori_loop(0, nch, step, 0)
        plsc.subcore_barrier()

        for shared, out, r in zip(shareds, outs, Rs):
            nzblk = r // CH

            def ocopy(t, carry, shared=shared, out=out, nzblk=nzblk, r=r):
                b = t * NS + sid

                @pl.when(b < nzblk)
                def _():
                    off = pl.multiple_of(b * CH, 8)
                    off2 = pl.multiple_of(cid * r + b * CH, 8)
                    pltpu.sync_copy(shared.at[pl.ds(off, CH)],
                                    out.at[pl.ds(off2, CH)])
                return carry

            lax.fori_loop(0, (nzblk + NS - 1) // NS, ocopy, 0)

    return pl.kernel(
        body,
        out_type=[jax.ShapeDtypeStruct((NC * r, D), jnp.float32)
                  for r in Rs],
        mesh=_mesh(),
        scratch_types=scratch,
        name=f"sc_{mode}_{K}_{R1}_{R2}",
    )


def _seg(table, src, dst, R):
    (o,) = _sc_pass(src.shape[0], "seg", R)(table, dst, src)
    return o.reshape(NC, R, D)


def _seg_w(table, src, dst, ew, R):
    (o,) = _sc_pass(src.shape[0], "segw", R)(table, dst, src, ew)
    return o.reshape(NC, R, D)


def _seg2(tab_a, tab_b, src, dst, R):
    oa, ob = _sc_pass(src.shape[0], "seg2", R)(tab_a, tab_b, dst, src)
    return oa.reshape(NC, R, D), ob.reshape(NC, R, D)


def _cnt(dst, R):
    (o,) = _sc_pass(dst.shape[0], "cnt", R)(dst)
    return o.reshape(NC, R, D)


def _cnt_w(dst, ew, R):
    (o,) = _sc_pass(dst.shape[0], "cntw", R)(dst, ew)
    return o.reshape(NC, R, D)


def _cnt2(dst1, dst2, R1, R2):
    o1, o2 = _sc_pass(dst1.shape[0], "cnt2", R1, R2)(dst1, dst2)
    return o1.reshape(NC, R1, D), o2.reshape(NC, R2, D)


def _gates(embed, Wh, bh, Wl, bl):
    BR = 2000

    def body(x_ref, wh_ref, bh_ref, wl_ref, bl_ref, h_ref, l_ref):
        x = x_ref[...]
        h_ref[...] = jax.nn.sigmoid(
            jnp.dot(x, wh_ref[...], preferred_element_type=jnp.float32)
            + bh_ref[...]) * x
        l_ref[...] = jax.nn.sigmoid(
            jnp.dot(x, wl_ref[...], preferred_element_type=jnp.float32)
            + bl_ref[...]) * x

    return pl.pallas_call(
        body,
        grid=(N // BR,),
        in_specs=[
            pl.BlockSpec((BR, D), lambda i: (i, 0)),
            pl.BlockSpec((D, D), lambda i: (0, 0)),
            pl.BlockSpec((1, D), lambda i: (0, 0)),
            pl.BlockSpec((D, D), lambda i: (0, 0)),
            pl.BlockSpec((1, D), lambda i: (0, 0)),
        ],
        out_specs=[
            pl.BlockSpec((BR, D), lambda i: (i, 0)),
            pl.BlockSpec((BR, D), lambda i: (i, 0)),
        ],
        out_shape=[
            jax.ShapeDtypeStruct((N, D), jnp.float32),
            jax.ShapeDtypeStruct((N, D), jnp.float32),
        ],
    )(embed, Wh, bh.reshape(1, D), Wl, bl.reshape(1, D))


def _eside(P, C, W):
    def body(p_ref, c_ref, w_ref, o_ref):
        s = p_ref[0] + p_ref[1]
        cnt = c_ref[0, :, 0:1] + c_ref[1, :, 0:1]
        o_ref[...] = jnp.dot(s, w_ref[...],
                             preferred_element_type=jnp.float32) / cnt

    return pl.pallas_call(
        body,
        out_shape=jax.ShapeDtypeStruct((E, D), jnp.float32),
    )(P, C, W)


@functools.lru_cache(maxsize=None)
def _nside_call(cc):
    BR = 2000

    def body(p_ref, c_ref, b_ref, x_ref, r_ref, xo_ref, ro_ref):
        s = p_ref[0] + p_ref[1]
        cnt = c_ref[0, :, 0:1] + c_ref[1, :, 0:1]
        dinv = jnp.where(cnt > 0, 1.0 / jnp.where(cnt > 0, cnt, 1.0), 0.0)
        t = dinv * s + b_ref[...]
        xn = jnp.where(t >= 0, t, 0.01 * t) + x_ref[...]
        xo_ref[...] = xn
        ro_ref[...] = r_ref[...] + cc * xn

    return pl.pallas_call(
        body,
        grid=(N // BR,),
        in_specs=[
            pl.BlockSpec((NC, BR, D), lambda i: (0, i, 0)),
            pl.BlockSpec((NC, BR, D), lambda i: (0, i, 0)),
            pl.BlockSpec((1, D), lambda i: (0, 0)),
            pl.BlockSpec((BR, D), lambda i: (i, 0)),
            pl.BlockSpec((BR, D), lambda i: (i, 0)),
        ],
        out_specs=[
            pl.BlockSpec((BR, D), lambda i: (i, 0)),
            pl.BlockSpec((BR, D), lambda i: (i, 0)),
        ],
        out_shape=[
            jax.ShapeDtypeStruct((N, D), jnp.float32),
            jax.ShapeDtypeStruct((N, D), jnp.float32),
        ],
    )


def _nside(P, C, b, X, res, cc):
    return _nside_call(cc)(P, C, b.reshape(1, D), X, res)


def _gcn_mm(X, W, C):
    def body(x_ref, w_ref, c_ref, xw_ref, y_ref):
        cnt = c_ref[0, :, 0:1] + c_ref[1, :, 0:1]
        dinv = lax.rsqrt(cnt + 1.0)
        xw = jnp.dot(x_ref[...], w_ref[...],
                     preferred_element_type=jnp.float32)
        xw_ref[...] = xw
        y_ref[...] = dinv * xw

    return pl.pallas_call(
        body,
        out_shape=[
            jax.ShapeDtypeStruct((X.shape[0], D), jnp.float32),
            jax.ShapeDtypeStruct((X.shape[0], D), jnp.float32),
        ],
    )(X, W, C)


@functools.lru_cache(maxsize=None)
def _gcn_comb_call(R, cc):
    def body(p_ref, xw_ref, x_ref, r_ref, c_ref, b_ref, xo_ref, ro_ref):
        cnt = c_ref[0, :, 0:1] + c_ref[1, :, 0:1]
        dinv = lax.rsqrt(cnt + 1.0)
        s = p_ref[0] + p_ref[1]
        t = dinv * s + (dinv * dinv) * xw_ref[...] + b_ref[...]
        xn = jnp.where(t >= 0, t, 0.01 * t) + x_ref[...]
        xo_ref[...] = xn
        ro_ref[...] = r_ref[...] + cc * xn

    return pl.pallas_call(
        body,
        out_shape=[
            jax.ShapeDtypeStruct((R, D), jnp.float32),
            jax.ShapeDtypeStruct((R, D), jnp.float32),
        ],
    )


def _gcn_comb(P, xw, X, res, C, b, cc):
    return _gcn_comb_call(X.shape[0], cc)(P, xw, X, res, C, b.reshape(1, D))


def kernel(diseases_embed, pros_embed, meds_embed, Wh, bh, Wl, bl,
           hg_W, hg_b, g_W, g_b, ddi_W, ddi_b, A_val, H_idx, A_idx, ddi_idx):
    embed = jnp.vstack([diseases_embed, pros_embed, meds_embed])
    node, he = H_idx[0], H_idx[1]
    eye = jnp.eye(D, dtype=jnp.float32)

    CB, CD = _cnt2(he, node, E, N)
    Cddi = _cnt(ddi_idx[1], N_MED)
    CA = _cnt_w(A_idx[1], A_val, E)

    h0, l0 = _gates(embed, Wh, bh, Wl, bl)

    X, res = meds_embed, meds_embed
    for i in range(L):
        xw, y = _gcn_mm(X, ddi_W[i], Cddi)
        Pg = _seg(y, ddi_idx[0], ddi_idx[1], N_MED)
        X, res = _gcn_comb(Pg, xw, X, res, Cddi, ddi_b[i], 1.0 / (i + 2))
    ddi_med = res

    Ph, Plr = _seg2(h0, l0, node, he, E)
    l_e = _eside(Plr, CB, eye)
    oe = _eside(Ph, CB, hg_W[0])
    Pv = _seg(oe, he, node, N)
    X, res = _nside(Pv, CD, hg_b[0], h0, h0, 0.5)
    P3 = _seg(X, node, he, E)
    oe = _eside(P3, CB, hg_W[1])
    Pv = _seg(oe, he, node, N)
    X, res = _nside(Pv, CD, hg_b[1], X, res, 1.0 / 3.0)
    h = res
    P5 = _seg(h, node, he, E)
    hr = _eside(P5, CB, eye)

    X, res = l_e, l_e
    for i in range(L):
        xw, y = _gcn_mm(X, g_W[i], CA)
        Pg = _seg_w(y, A_idx[0], A_idx[1], A_val, E)
        X, res = _gcn_comb(Pg, xw, X, res, CA, g_b[i], 1.0 / (i + 2))
    lin = res

    return (hr, lin, h[:N_DIS], h[N_DIS:N_DIS + N_PRO],
            h[N_DIS + N_PRO:], ddi_med)


# speedup vs baseline: 15.2352x; 2.2616x over previous
import functools

import jax
import jax.numpy as jnp
from jax import lax
from jax.experimental import pallas as pl
from jax.experimental.pallas import tpu as pltpu
from jax.experimental.pallas import tpu_sc as plsc

N_DIS, N_PRO, N_MED = 4000, 4000, 2000
N = N_DIS + N_PRO + N_MED
D = 128
E = 2000
L = 2

NC, NS, LANES = 2, 16, 16
NW = NC * NS
CH = 80


def _mesh():
    return plsc.VectorSubcoreMesh(
        core_axis_name="c", subcore_axis_name="s",
        num_cores=NC, num_subcores=NS)


@functools.lru_cache(maxsize=None)
def _sc_pass(K, mode, R1, R2=0):
    per_w = K // NW
    assert per_w * NW == K and per_w % CH == 0
    nch = per_w // CH
    assert nch % 2 == 1 and nch >= 3

    two_acc = mode in ("seg2", "cnt2")
    gather = mode in ("seg", "segw", "seg2")
    weighted = mode in ("segw", "cntw")
    ones = mode in ("cnt", "cnt2")
    Rs = [R1, R2] if mode == "cnt2" else ([R1, R1] if two_acc else [R1])
    ndst = 2 if mode == "cnt2" else 1
    ntab = 2 if mode == "seg2" else (1 if gather else 0)
    NSL = 2
    nrows = NSL * ntab if gather else 1

    scratch = (
        [pltpu.VMEM((CH,), jnp.int32) for _ in range(ndst * NSL)]
        + [pltpu.VMEM((CH,), jnp.int32) for _ in range(NSL if gather else 0)]
        + [pltpu.VMEM((CH,), jnp.float32) for _ in range(NSL if weighted else 0)]
        + [pltpu.VMEM((CH, D), jnp.float32) for _ in range(nrows)]
        + [pltpu.VMEM_SHARED((r, D), jnp.float32) for r in Rs]
        + [pltpu.SemaphoreType.DMA for _ in range(NSL)]
        + [pltpu.SemaphoreType.DMA for _ in range(NSL * ntab)]
    )

    def body(*refs):
        refs = list(refs)
        tabs = [refs.pop(0) for _ in range(ntab)]
        dsth = [refs.pop(0) for _ in range(ndst)]
        srch = refs.pop(0) if gather else None
        ewh = refs.pop(0) if weighted else None
        outs = [refs.pop(0) for _ in range(len(Rs))]
        dstv = [[refs.pop(0) for _ in range(ndst)] for _ in range(NSL)]
        srcv = [refs.pop(0) for _ in range(NSL)] if gather else None
        ewv = [refs.pop(0) for _ in range(NSL)] if weighted else None
        rows = ([[refs.pop(0) for _ in range(ntab)] for _ in range(NSL)]
                if gather else [[refs.pop(0)]])
        shareds = [refs.pop(0) for _ in range(len(Rs))]
        isem = [refs.pop(0) for _ in range(NSL)]
        gsem = [[refs.pop(0) for _ in range(ntab)] for _ in range(NSL)]

        cid = lax.axis_index("c")
        sid = lax.axis_index("s")
        wid = sid * NC + cid
        base = wid * per_w

        zv = jnp.zeros((LANES,), jnp.float32)

        def zfill(j, carry):
            for pp in range(D // LANES):
                rows[0][0][j, pl.ds(pp * LANES, LANES)] = zv
            return carry

        lax.fori_loop(0, CH, zfill, 0)

        for shared, r in zip(shareds, Rs):
            nzblk = r // CH

            def zcopy(t, carry, shared=shared, nzblk=nzblk):
                b = t * NS + sid

                @pl.when(b < nzblk)
                def _():
                    off = pl.multiple_of(b * CH, 8)
                    pltpu.sync_copy(rows[0][0], shared.at[pl.ds(off, CH)])
                return carry

            lax.fori_loop(0, (nzblk + NS - 1) // NS, zcopy, 0)

        if ones:
            ov = jnp.ones((LANES,), jnp.float32)

            def ofill(j, carry):
                for pp in range(D // LANES):
                    rows[0][0][j, pl.ds(pp * LANES, LANES)] = ov
                return carry

            lax.fori_loop(0, CH, ofill, 0)

        plsc.subcore_barrier()

        def idescs(slot, t):
            off = base + t * CH
            ds = [pltpu.make_async_copy(dh.at[pl.ds(off, CH)], dv, isem[slot])
                  for dh, dv in zip(dsth, dstv[slot])]
            if gather:
                ds.append(pltpu.make_async_copy(
                    srch.at[pl.ds(off, CH)], srcv[slot], isem[slot]))
            if weighted:
                ds.append(pltpu.make_async_copy(
                    ewh.at[pl.ds(off, CH)], ewv[slot], isem[slot]))
            return ds

        def istart(slot, t):
            @pl.when(t < nch)
            def _():
                for dsc in idescs(slot, t):
                    dsc.start()

        def iwait(slot, t):
            for dsc in idescs(slot, t):
                dsc.wait()

        def gstart(slot):
            for ti in range(ntab):
                pltpu.async_copy(tabs[ti].at[srcv[slot]],
                                 rows[slot][ti], gsem[slot][ti])

        def gwait(slot):
            for ti in range(ntab):
                pltpu.make_async_copy(tabs[ti].at[srcv[slot]],
                                      rows[slot][ti], gsem[slot][ti]).wait()

        def do_scale(slot):
            if not weighted:
                return
            rb = rows[slot][0] if gather else rows[0][0]
            if not gather:
                ov = jnp.ones((LANES,), jnp.float32)

                def refill(j, c2):
                    for pp in range(D // LANES):
                        rb[j, pl.ds(pp * LANES, LANES)] = ov
                    return c2

                lax.fori_loop(0, CH, refill, 0)

            def scale(q, c2):
                wchunk = ewv[slot][pl.ds(q * LANES, LANES)]
                for jj in range(LANES):
                    w = wchunk[jnp.full((LANES,), jj, jnp.int32)]
                    j = q * LANES + jj
                    for pp in range(D // LANES):
                        sl = pl.ds(pp * LANES, LANES)
                        rb[j, sl] = rb[j, sl] * w
                return c2

            lax.fori_loop(0, CH // LANES, scale, 0)

        def do_scatter(slot):
            if mode == "cnt2":
                pltpu.sync_copy(rows[0][0],
                                shareds[0].at[dstv[slot][0]], add=True)
                pltpu.sync_copy(rows[0][0],
                                shareds[1].at[dstv[slot][1]], add=True)
            elif gather:
                for ti in range(ntab):
                    pltpu.sync_copy(rows[slot][ti],
                                    shareds[ti].at[dstv[slot][0]], add=True)
            else:
                pltpu.sync_copy(rows[0][0],
                                shareds[0].at[dstv[slot][0]], add=True)

        istart(0, 0)
        istart(1, 1)
        iwait(0, 0)
        if gather:
            gstart(0)

        npairs = (nch - 1) // 2

        def pbody(t2, carry):
            a = 2 * t2
            iwait(1, a + 1)
            if gather:
                gstart(1)
            istart(0, a + 2)
            if gather:
                gwait(0)
            do_scale(0)
            do_scatter(0)
            iwait(0, a + 2)
            if gather:
                gstart(0)
            istart(1, a + 3)
            if gather:
                gwait(1)
            do_scale(1)
            do_scatter(1)
            return carry

        lax.fori_loop(0, npairs, pbody, 0)
        if gather:
            gwait(0)
        do_scale(0)
        do_scatter(0)

        plsc.subcore_barrier()

        for shared, out, r in zip(shareds, outs, Rs):
            nzblk = r // CH

            def ocopy(t, carry, shared=shared, out=out, nzblk=nzblk, r=r):
                b = t * NS + sid

                @pl.when(b < nzblk)
                def _():
                    off = pl.multiple_of(b * CH, 8)
                    off2 = pl.multiple_of(cid * r + b * CH, 8)
                    pltpu.sync_copy(shared.at[pl.ds(off, CH)],
                                    out.at[pl.ds(off2, CH)])
                return carry

            lax.fori_loop(0, (nzblk + NS - 1) // NS, ocopy, 0)

    return pl.kernel(
        body,
        out_type=[jax.ShapeDtypeStruct((NC * r, D), jnp.float32)
                  for r in Rs],
        mesh=_mesh(),
        scratch_types=scratch,
        name=f"sc_{mode}_{K}_{R1}_{R2}",
    )


def _seg(table, src, dst, R):
    (o,) = _sc_pass(src.shape[0], "seg", R)(table, dst, src)
    return o.reshape(NC, R, D)


def _seg_w(table, src, dst, ew, R):
    (o,) = _sc_pass(src.shape[0], "segw", R)(table, dst, src, ew)
    return o.reshape(NC, R, D)


def _seg2(tab_a, tab_b, src, dst, R):
    oa, ob = _sc_pass(src.shape[0], "seg2", R)(tab_a, tab_b, dst, src)
    return oa.reshape(NC, R, D), ob.reshape(NC, R, D)


def _cnt(dst, R):
    (o,) = _sc_pass(dst.shape[0], "cnt", R)(dst)
    return o.reshape(NC, R, D)


def _cnt_w(dst, ew, R):
    (o,) = _sc_pass(dst.shape[0], "cntw", R)(dst, ew)
    return o.reshape(NC, R, D)


def _cnt2(dst1, dst2, R1, R2):
    o1, o2 = _sc_pass(dst1.shape[0], "cnt2", R1, R2)(dst1, dst2)
    return o1.reshape(NC, R1, D), o2.reshape(NC, R2, D)


def _gates(embed, Wh, bh, Wl, bl):
    BR = 2000

    def body(x_ref, wh_ref, bh_ref, wl_ref, bl_ref, h_ref, l_ref):
        x = x_ref[...]
        h_ref[...] = jax.nn.sigmoid(
            jnp.dot(x, wh_ref[...], preferred_element_type=jnp.float32)
            + bh_ref[...]) * x
        l_ref[...] = jax.nn.sigmoid(
            jnp.dot(x, wl_ref[...], preferred_element_type=jnp.float32)
            + bl_ref[...]) * x

    return pl.pallas_call(
        body,
        grid=(N // BR,),
        in_specs=[
            pl.BlockSpec((BR, D), lambda i: (i, 0)),
            pl.BlockSpec((D, D), lambda i: (0, 0)),
            pl.BlockSpec((1, D), lambda i: (0, 0)),
            pl.BlockSpec((D, D), lambda i: (0, 0)),
            pl.BlockSpec((1, D), lambda i: (0, 0)),
        ],
        out_specs=[
            pl.BlockSpec((BR, D), lambda i: (i, 0)),
            pl.BlockSpec((BR, D), lambda i: (i, 0)),
        ],
        out_shape=[
            jax.ShapeDtypeStruct((N, D), jnp.float32),
            jax.ShapeDtypeStruct((N, D), jnp.float32),
        ],
    )(embed, Wh, bh.reshape(1, D), Wl, bl.reshape(1, D))


def _eside(P, C, W):
    def body(p_ref, c_ref, w_ref, o_ref):
        s = p_ref[0] + p_ref[1]
        cnt = c_ref[0, :, 0:1] + c_ref[1, :, 0:1]
        o_ref[...] = jnp.dot(s, w_ref[...],
                             preferred_element_type=jnp.float32) / cnt

    return pl.pallas_call(
        body,
        out_shape=jax.ShapeDtypeStruct((E, D), jnp.float32),
    )(P, C, W)


@functools.lru_cache(maxsize=None)
def _nside_call(cc):
    BR = 2000

    def body(p_ref, c_ref, b_ref, x_ref, r_ref, xo_ref, ro_ref):
        s = p_ref[0] + p_ref[1]
        cnt = c_ref[0, :, 0:1] + c_ref[1, :, 0:1]
        dinv = jnp.where(cnt > 0, 1.0 / jnp.where(cnt > 0, cnt, 1.0), 0.0)
        t = dinv * s + b_ref[...]
        xn = jnp.where(t >= 0, t, 0.01 * t) + x_ref[...]
        xo_ref[...] = xn
        ro_ref[...] = r_ref[...] + cc * xn

    return pl.pallas_call(
        body,
        grid=(N // BR,),
        in_specs=[
            pl.BlockSpec((NC, BR, D), lambda i: (0, i, 0)),
            pl.BlockSpec((NC, BR, D), lambda i: (0, i, 0)),
            pl.BlockSpec((1, D), lambda i: (0, 0)),
            pl.BlockSpec((BR, D), lambda i: (i, 0)),
            pl.BlockSpec((BR, D), lambda i: (i, 0)),
        ],
        out_specs=[
            pl.BlockSpec((BR, D), lambda i: (i, 0)),
            pl.BlockSpec((BR, D), lambda i: (i, 0)),
        ],
        out_shape=[
            jax.ShapeDtypeStruct((N, D), jnp.float32),
            jax.ShapeDtypeStruct((N, D), jnp.float32),
        ],
    )


def _nside(P, C, b, X, res, cc):
    return _nside_call(cc)(P, C, b.reshape(1, D), X, res)


def _gcn_mm(X, W, C):
    def body(x_ref, w_ref, c_ref, xw_ref, y_ref):
        cnt = c_ref[0, :, 0:1] + c_ref[1, :, 0:1]
        dinv = lax.rsqrt(cnt + 1.0)
        xw = jnp.dot(x_ref[...], w_ref[...],
                     preferred_element_type=jnp.float32)
        xw_ref[...] = xw
        y_ref[...] = dinv * xw

    return pl.pallas_call(
        body,
        out_shape=[
            jax.ShapeDtypeStruct((X.shape[0], D), jnp.float32),
            jax.ShapeDtypeStruct((X.shape[0], D), jnp.float32),
        ],
    )(X, W, C)


@functools.lru_cache(maxsize=None)
def _gcn_comb_call(R, cc):
    def body(p_ref, xw_ref, x_ref, r_ref, c_ref, b_ref, xo_ref, ro_ref):
        cnt = c_ref[0, :, 0:1] + c_ref[1, :, 0:1]
        dinv = lax.rsqrt(cnt + 1.0)
        s = p_ref[0] + p_ref[1]
        t = dinv * s + (dinv * dinv) * xw_ref[...] + b_ref[...]
        xn = jnp.where(t >= 0, t, 0.01 * t) + x_ref[...]
        xo_ref[...] = xn
        ro_ref[...] = r_ref[...] + cc * xn

    return pl.pallas_call(
        body,
        out_shape=[
            jax.ShapeDtypeStruct((R, D), jnp.float32),
            jax.ShapeDtypeStruct((R, D), jnp.float32),
        ],
    )


def _gcn_comb(P, xw, X, res, C, b, cc):
    return _gcn_comb_call(X.shape[0], cc)(P, xw, X, res, C, b.reshape(1, D))


def kernel(diseases_embed, pros_embed, meds_embed, Wh, bh, Wl, bl,
           hg_W, hg_b, g_W, g_b, ddi_W, ddi_b, A_val, H_idx, A_idx, ddi_idx):
    embed = jnp.vstack([diseases_embed, pros_embed, meds_embed])
    node, he = H_idx[0], H_idx[1]
    eye = jnp.eye(D, dtype=jnp.float32)

    CB, CD = _cnt2(he, node, E, N)
    Cddi = _cnt(ddi_idx[1], N_MED)
    CA = _cnt_w(A_idx[1], A_val, E)

    h0, l0 = _gates(embed, Wh, bh, Wl, bl)

    X, res = meds_embed, meds_embed
    for i in range(L):
        xw, y = _gcn_mm(X, ddi_W[i], Cddi)
        Pg = _seg(y, ddi_idx[0], ddi_idx[1], N_MED)
        X, res = _gcn_comb(Pg, xw, X, res, Cddi, ddi_b[i], 1.0 / (i + 2))
    ddi_med = res

    Ph, Plr = _seg2(h0, l0, node, he, E)
    l_e = _eside(Plr, CB, eye)
    oe = _eside(Ph, CB, hg_W[0])
    Pv = _seg(oe, he, node, N)
    X, res = _nside(Pv, CD, hg_b[0], h0, h0, 0.5)
    P3 = _seg(X, node, he, E)
    oe = _eside(P3, CB, hg_W[1])
    Pv = _seg(oe, he, node, N)
    X, res = _nside(Pv, CD, hg_b[1], X, res, 1.0 / 3.0)
    h = res
    P5 = _seg(h, node, he, E)
    hr = _eside(P5, CB, eye)

    X, res = l_e, l_e
    for i in range(L):
        xw, y = _gcn_mm(X, g_W[i], CA)
        Pg = _seg_w(y, A_idx[0], A_idx[1], A_val, E)
        X, res = _gcn_comb(Pg, xw, X, res, CA, g_b[i], 1.0 / (i + 2))
    lin = res

    return (hr, lin, h[:N_DIS], h[N_DIS:N_DIS + N_PRO],
            h[N_DIS + N_PRO:], ddi_med)
